# baseline (speedup 1.0000x reference)
import jax
import jax.numpy as jnp
from jax.experimental import pallas as pl
from jax.experimental.pallas import tpu as pltpu

_B = 1024
_H = 50
_K = 64
_C = 256
_W = _H * _K // 128
_CR = _C * _W
_R = _B // _C
_Q = 4096 // _B


def _probe_body(x_ref, oe_ref, oo_ref, acc_ref):
    q = pl.program_id(1)

    lane = jax.lax.broadcasted_iota(jnp.int32, (128, 128), 0)
    col = jax.lax.broadcasted_iota(jnp.int32, (128, 128), 1)
    sel = (col == (lane // _K) * _K).astype(jnp.float32)

    a = jnp.abs(x_ref[...])
    s = jax.lax.dot_general(
        a, sel, (((1,), (0,)), ((), ())),
        preferred_element_type=jnp.float32,
    )
    rz = (s == 0.0).astype(jnp.float32)

    @pl.when(q == 0)
    def _init():
        acc_ref[...] = rz

    @pl.when(q > 0)
    def _acc():
        acc_ref[...] += rz

    @pl.when(q == _Q - 1)
    def _emit():
        g = acc_ref[...].reshape(_C, _W, 128)
        oe_ref[...] = g[:, :, 0]
        oo_ref[...] = g[:, :, _K]


def kernel(x, seen):
    x0 = x[:, 0].reshape(4096 * _W, 128)
    oe, oo = pl.pallas_call(
        _probe_body,
        grid=(_R, _Q),
        in_specs=[pl.BlockSpec((_CR, 128), lambda r, q: (r + _R * q, 0))],
        out_specs=[
            pl.BlockSpec((_C, _W), lambda r, q: (r, 0)),
            pl.BlockSpec((_C, _W), lambda r, q: (r, 0)),
        ],
        out_shape=[
            jax.ShapeDtypeStruct((_B, _W), jnp.float32),
            jax.ShapeDtypeStruct((_B, _W), jnp.float32),
        ],
        scratch_shapes=[pltpu.VMEM((_CR, 128), jnp.float32)],
    )(x0)
    buf = jnp.stack([oe, oo], axis=-1).reshape(_B, _H)
    return (x, seen + buf)
